# baseline (speedup 1.0000x reference)
import functools

import jax
import jax.numpy as jnp
from jax import lax
from jax.experimental import pallas as pl
from jax.experimental.pallas import tpu as pltpu, tpu_sc as plsc

VOCAB = 1000000
EMBED = 16
N = 1638400
B = 16384
HIDDEN = 128
NUM_CLASS = 10

NC = 2
NS = 16
NW = NC * NS
PER_W = N // NW
CHUNK = 2560
CPW = PER_W // CHUNK
SUB = 128
SUBS = CHUNK // SUB
ROWS_PER_TILE = B // NS
ZROWS = 128

BLK = 1024
N128 = N // 128


def _sc_pool_body(tags_hbm, seg_hbm, tab_hbm, out_hbm,
                  idx0, idx1, seg0, seg1, rows0, rows1, zero_v, acc_sh,
                  gsem, ssem0, ssem1):
    cid = lax.axis_index("c")
    sid = lax.axis_index("s")
    wid = cid * NS + sid
    idx = (idx0, idx1)
    seg = (seg0, seg1)
    rows = (rows0, rows1)
    ssem = (ssem0, ssem1)

    def zbody(i, carry):
        zero_v[i] = jnp.zeros((EMBED,), jnp.float32)
        return carry
    lax.fori_loop(0, ZROWS, zbody, 0)
    for z in range(ROWS_PER_TILE // ZROWS):
        pltpu.sync_copy(zero_v,
                        acc_sh.at[pl.ds(sid * ROWS_PER_TILE + z * ZROWS, ZROWS)])
    plsc.subcore_barrier()

    def load(g, p):
        base = wid * PER_W + g * CHUNK
        pltpu.sync_copy(tags_hbm.at[pl.ds(base, CHUNK)], idx[p])
        pltpu.sync_copy(seg_hbm.at[pl.ds(base // 128, SUBS)], seg[p])
        ref = idx[p]

        def tbody(k, carry):
            v = ref[pl.ds(k * 16, 16)]
            ref[pl.ds(k * 16, 16)] = ((v & -8192) | ((v & 1023) << 3)
                                      | ((v >> 10) & 7))
            return carry
        lax.fori_loop(0, CHUNK // 16, tbody, 0, unroll=8)

    def gather(p):
        return pltpu.make_async_copy(tab_hbm.at[idx[p]], rows[p], gsem)

    def scatter_start(p):
        for j in range(SUBS):
            pltpu.make_async_copy(rows[p].at[pl.ds(j * SUB, SUB)],
                                  acc_sh.at[seg[p].at[j]], ssem[p]).start(add=True)

    def scatter_drain(p):
        for j in range(SUBS):
            pltpu.make_async_copy(rows[p].at[pl.ds(j * SUB, SUB)],
                                  acc_sh.at[seg[p].at[j]], ssem[p]).wait()

    load(0, 0)
    gather(0).start()

    def body(g, carry):
        for p in (0, 1):
            @pl.when(g % 2 == p)
            def _():
                gather(p).wait()

                @pl.when(g > 0)
                def _():
                    scatter_drain(1 - p)

                @pl.when(g + 1 < CPW)
                def _():
                    load(g + 1, 1 - p)
                    gather(1 - p).start()
                scatter_start(p)
        return carry
    lax.fori_loop(0, CPW, body, 0)
    scatter_drain((CPW - 1) % 2)

    plsc.subcore_barrier()
    pltpu.sync_copy(
        acc_sh.at[pl.ds(sid * ROWS_PER_TILE, ROWS_PER_TILE)],
        out_hbm.at[pl.ds(cid * B + sid * ROWS_PER_TILE, ROWS_PER_TILE)])


TBK = 8192
TGRID = (VOCAB + TBK - 1) // TBK
VOCABP = TGRID * TBK


def _tr_body(xt_ref, out_ref):
    eye = (lax.broadcasted_iota(jnp.int32, (EMBED, EMBED), 0)
           == lax.broadcasted_iota(jnp.int32, (EMBED, EMBED), 1)
           ).astype(jnp.float32)
    for u in range(8):
        out_ref[:, 16 * u:16 * (u + 1)] = lax.dot_general(
            xt_ref[:, 1024 * u:1024 * (u + 1)], eye,
            dimension_numbers=(((0,), (0,)), ((), ())),
            precision=lax.Precision.HIGHEST,
            preferred_element_type=jnp.float32)


def _mlp_body(pa_ref, pb_ref, lab_ref, w1_ref, b1_ref, w2_ref, b2_ref,
              out_ref, loss_ref):
    i = pl.program_id(0)
    pooled = pa_ref[...] + pb_ref[...]
    h = jnp.maximum(
        jnp.dot(pooled, w1_ref[...], preferred_element_type=jnp.float32)
        + b1_ref[...], 0.0)
    logits = jnp.dot(h, w2_ref[...],
                     preferred_element_type=jnp.float32) + b2_ref[...]
    out_ref[...] = logits
    lab = lab_ref[...]
    cls = lax.broadcasted_iota(jnp.int32, (BLK, NUM_CLASS), 1)
    logit_lab = jnp.sum(jnp.where(cls == lab, logits, 0.0), axis=1,
                        keepdims=True)
    m = jnp.max(logits, axis=1, keepdims=True)
    lse = jnp.log(jnp.sum(jnp.exp(logits - m), axis=1, keepdims=True)) + m
    part = (jnp.sum(lse - logit_lab) / B).reshape(1, 1)

    @pl.when(i == 0)
    def _():
        loss_ref[...] = jnp.zeros_like(loss_ref)
    loss_ref[...] += part


def kernel(node_tags, segment_ids, labels, table, W1, b1, W2, b2):
    tab_lin = pl.pallas_call(
        _tr_body,
        grid=(TGRID,),
        in_specs=[pl.BlockSpec((EMBED, TBK), lambda i: (0, i))],
        out_specs=pl.BlockSpec((TBK // 8, 128), lambda i: (i, 0)),
        out_shape=jax.ShapeDtypeStruct((VOCABP // 8, 128), jnp.float32),
        compiler_params=pltpu.CompilerParams(
            fuse_transposed_lhs_in_matmul=True),
    )(table.T).reshape(VOCABP, EMBED)

    mesh = plsc.VectorSubcoreMesh(core_axis_name="c", subcore_axis_name="s",
                                  num_cores=NC, num_subcores=NS)
    sc_pool = pl.kernel(
        _sc_pool_body,
        out_type=jax.ShapeDtypeStruct((NC * B, EMBED), jnp.float32),
        mesh=mesh,
        scratch_types=[
            pltpu.VMEM((CHUNK,), jnp.int32),
            pltpu.VMEM((CHUNK,), jnp.int32),
            pltpu.VMEM((SUBS, SUB), jnp.int32),
            pltpu.VMEM((SUBS, SUB), jnp.int32),
            pltpu.VMEM((CHUNK, EMBED), jnp.float32),
            pltpu.VMEM((CHUNK, EMBED), jnp.float32),
            pltpu.VMEM((ZROWS, EMBED), jnp.float32),
            pltpu.VMEM_SHARED((B, EMBED), jnp.float32),
            pltpu.SemaphoreType.DMA,
            pltpu.SemaphoreType.DMA,
            pltpu.SemaphoreType.DMA,
        ],
        compiler_params=pltpu.CompilerParams(use_tc_tiling_on_sc=False),
    )
    part = sc_pool(node_tags, segment_ids.reshape(N128, 128), tab_lin)

    grid = (B // BLK,)
    logits, loss2 = pl.pallas_call(
        _mlp_body,
        grid=grid,
        in_specs=[
            pl.BlockSpec((BLK, EMBED), lambda i: (i, 0)),
            pl.BlockSpec((BLK, EMBED), lambda i: (i + B // BLK, 0)),
            pl.BlockSpec((BLK, 1), lambda i: (i, 0)),
            pl.BlockSpec((EMBED, HIDDEN), lambda i: (0, 0)),
            pl.BlockSpec((1, HIDDEN), lambda i: (0, 0)),
            pl.BlockSpec((HIDDEN, NUM_CLASS), lambda i: (0, 0)),
            pl.BlockSpec((1, NUM_CLASS), lambda i: (0, 0)),
        ],
        out_specs=[
            pl.BlockSpec((BLK, NUM_CLASS), lambda i: (i, 0)),
            pl.BlockSpec((1, 1), lambda i: (0, 0)),
        ],
        out_shape=[
            jax.ShapeDtypeStruct((B, NUM_CLASS), jnp.float32),
            jax.ShapeDtypeStruct((1, 1), jnp.float32),
        ],
    )(part, part, labels.reshape(B, 1), W1, b1.reshape(1, HIDDEN),
      W2, b2.reshape(1, NUM_CLASS))
    return logits, loss2[0, 0]


# speedup vs baseline: 2.4477x; 2.4477x over previous
import functools

import jax
import jax.numpy as jnp
from jax import lax
from jax.experimental import pallas as pl
from jax.experimental.pallas import tpu as pltpu, tpu_sc as plsc

VOCAB = 1000000
EMBED = 16
N = 1638400
B = 16384
HIDDEN = 128
NUM_CLASS = 10

NC = 2
NS = 16
NW = NC * NS
PER_W = N // NW
CHUNK = 2560
CPW = PER_W // CHUNK
SUB = 128
SUBS = CHUNK // SUB
ROWS_PER_TILE = B // NS
ZROWS = 128

BLK = 1024
N128 = N // 128


def _sc_pool_body(tags_hbm, seg_hbm, tab_hbm, out_hbm,
                  idx0, idx1, seg0, seg1, rows0, rows1, zero_v, acc_sh,
                  gsem, ssem0, ssem1):
    cid = lax.axis_index("c")
    sid = lax.axis_index("s")
    wid = cid * NS + sid
    idx = (idx0, idx1)
    seg = (seg0, seg1)
    rows = (rows0, rows1)
    ssem = (ssem0, ssem1)

    def zbody(i, carry):
        zero_v[i] = jnp.zeros((EMBED,), jnp.float32)
        return carry
    lax.fori_loop(0, ZROWS, zbody, 0)
    for z in range(ROWS_PER_TILE // ZROWS):
        pltpu.sync_copy(zero_v,
                        acc_sh.at[pl.ds(sid * ROWS_PER_TILE + z * ZROWS, ZROWS)])
    plsc.subcore_barrier()

    def load(g, p):
        base = wid * PER_W + g * CHUNK
        pltpu.sync_copy(tags_hbm.at[pl.ds(base, CHUNK)], idx[p])
        pltpu.sync_copy(seg_hbm.at[pl.ds(base // 128, SUBS)], seg[p])
        ref = idx[p]

        def tbody(k, carry):
            v = ref[pl.ds(k * 16, 16)]
            ref[pl.ds(k * 16, 16)] = ((v & -8192) | ((v & 1023) << 3)
                                      | ((v >> 10) & 7))
            return carry
        lax.fori_loop(0, CHUNK // 16, tbody, 0, unroll=8)

    def gather(p):
        return pltpu.make_async_copy(tab_hbm.at[idx[p]], rows[p], gsem)

    def scatter_start(p):
        for j in range(SUBS):
            pltpu.make_async_copy(rows[p].at[pl.ds(j * SUB, SUB)],
                                  acc_sh.at[seg[p].at[j]], ssem[p]).start(add=True)

    def scatter_drain(p):
        for j in range(SUBS):
            pltpu.make_async_copy(rows[p].at[pl.ds(j * SUB, SUB)],
                                  acc_sh.at[seg[p].at[j]], ssem[p]).wait()

    load(0, 0)
    gather(0).start()

    def body(g, carry):
        for p in (0, 1):
            @pl.when(g % 2 == p)
            def _():
                gather(p).wait()

                @pl.when(g > 0)
                def _():
                    scatter_drain(1 - p)

                @pl.when(g + 1 < CPW)
                def _():
                    load(g + 1, 1 - p)
                    gather(1 - p).start()
                scatter_start(p)
        return carry
    lax.fori_loop(0, CPW, body, 0)
    scatter_drain((CPW - 1) % 2)

    plsc.subcore_barrier()
    pltpu.sync_copy(
        acc_sh.at[pl.ds(sid * ROWS_PER_TILE, ROWS_PER_TILE)],
        out_hbm.at[pl.ds(cid * B + sid * ROWS_PER_TILE, ROWS_PER_TILE)])


TBK = 8192
TGRID = (VOCAB + TBK - 1) // TBK
VOCABP = TGRID * TBK


def _tr_body(xt_ref, out_ref):
    y = jnp.concatenate(
        [xt_ref[:, 1024 * u:1024 * (u + 1)] for u in range(8)], axis=0)
    out_ref[...] = y.T


def _mlp_body(pa_ref, pb_ref, lab_ref, w1_ref, b1_ref, w2_ref, b2_ref,
              out_ref, loss_ref):
    i = pl.program_id(0)
    pooled = pa_ref[...] + pb_ref[...]
    h = jnp.maximum(
        jnp.dot(pooled, w1_ref[...], preferred_element_type=jnp.float32)
        + b1_ref[...], 0.0)
    logits = jnp.dot(h, w2_ref[...],
                     preferred_element_type=jnp.float32) + b2_ref[...]
    out_ref[...] = logits
    lab = lab_ref[...]
    cls = lax.broadcasted_iota(jnp.int32, (BLK, NUM_CLASS), 1)
    logit_lab = jnp.sum(jnp.where(cls == lab, logits, 0.0), axis=1,
                        keepdims=True)
    m = jnp.max(logits, axis=1, keepdims=True)
    lse = jnp.log(jnp.sum(jnp.exp(logits - m), axis=1, keepdims=True)) + m
    part = (jnp.sum(lse - logit_lab) / B).reshape(1, 1)

    @pl.when(i == 0)
    def _():
        loss_ref[...] = jnp.zeros_like(loss_ref)
    loss_ref[...] += part


def kernel(node_tags, segment_ids, labels, table, W1, b1, W2, b2):
    tab_lin = pl.pallas_call(
        _tr_body,
        grid=(TGRID,),
        in_specs=[pl.BlockSpec((EMBED, TBK), lambda i: (0, i))],
        out_specs=pl.BlockSpec((TBK // 8, 128), lambda i: (i, 0)),
        out_shape=jax.ShapeDtypeStruct((VOCABP // 8, 128), jnp.float32),
    )(table.T).reshape(VOCABP, EMBED)

    mesh = plsc.VectorSubcoreMesh(core_axis_name="c", subcore_axis_name="s",
                                  num_cores=NC, num_subcores=NS)
    sc_pool = pl.kernel(
        _sc_pool_body,
        out_type=jax.ShapeDtypeStruct((NC * B, EMBED), jnp.float32),
        mesh=mesh,
        scratch_types=[
            pltpu.VMEM((CHUNK,), jnp.int32),
            pltpu.VMEM((CHUNK,), jnp.int32),
            pltpu.VMEM((SUBS, SUB), jnp.int32),
            pltpu.VMEM((SUBS, SUB), jnp.int32),
            pltpu.VMEM((CHUNK, EMBED), jnp.float32),
            pltpu.VMEM((CHUNK, EMBED), jnp.float32),
            pltpu.VMEM((ZROWS, EMBED), jnp.float32),
            pltpu.VMEM_SHARED((B, EMBED), jnp.float32),
            pltpu.SemaphoreType.DMA,
            pltpu.SemaphoreType.DMA,
            pltpu.SemaphoreType.DMA,
        ],
        compiler_params=pltpu.CompilerParams(use_tc_tiling_on_sc=False),
    )
    part = sc_pool(node_tags, segment_ids.reshape(N128, 128), tab_lin)

    grid = (B // BLK,)
    logits, loss2 = pl.pallas_call(
        _mlp_body,
        grid=grid,
        in_specs=[
            pl.BlockSpec((BLK, EMBED), lambda i: (i, 0)),
            pl.BlockSpec((BLK, EMBED), lambda i: (i + B // BLK, 0)),
            pl.BlockSpec((BLK, 1), lambda i: (i, 0)),
            pl.BlockSpec((EMBED, HIDDEN), lambda i: (0, 0)),
            pl.BlockSpec((1, HIDDEN), lambda i: (0, 0)),
            pl.BlockSpec((HIDDEN, NUM_CLASS), lambda i: (0, 0)),
            pl.BlockSpec((1, NUM_CLASS), lambda i: (0, 0)),
        ],
        out_specs=[
            pl.BlockSpec((BLK, NUM_CLASS), lambda i: (i, 0)),
            pl.BlockSpec((1, 1), lambda i: (0, 0)),
        ],
        out_shape=[
            jax.ShapeDtypeStruct((B, NUM_CLASS), jnp.float32),
            jax.ShapeDtypeStruct((1, 1), jnp.float32),
        ],
    )(part, part, labels.reshape(B, 1), W1, b1.reshape(1, HIDDEN),
      W2, b2.reshape(1, NUM_CLASS))
    return logits, loss2[0, 0]


# speedup vs baseline: 2.5265x; 1.0322x over previous
import functools

import jax
import jax.numpy as jnp
from jax import lax
from jax.experimental import pallas as pl
from jax.experimental.pallas import tpu as pltpu, tpu_sc as plsc

VOCAB = 1000000
EMBED = 16
N = 1638400
B = 16384
HIDDEN = 128
NUM_CLASS = 10

NC = 2
NS = 16
NW = NC * NS
PER_W = N // NW
CHUNK = 2560
CPW = PER_W // CHUNK
SUB = 128
SUBS = CHUNK // SUB
ROWS_PER_TILE = B // NS
ZROWS = 128

BLK = 1024
N128 = N // 128


def _sc_pool_body(tags_hbm, seg_hbm, tab_hbm, out_hbm,
                  idx0, idx1, seg0, seg1, rows0, rows1, zero_v, acc_sh,
                  gsem, ssem0, ssem1):
    cid = lax.axis_index("c")
    sid = lax.axis_index("s")
    wid = cid * NS + sid
    idx = (idx0, idx1)
    seg = (seg0, seg1)
    rows = (rows0, rows1)
    ssem = (ssem0, ssem1)

    def zbody(i, carry):
        zero_v[i] = jnp.zeros((EMBED,), jnp.float32)
        return carry
    lax.fori_loop(0, ZROWS, zbody, 0)
    for z in range(ROWS_PER_TILE // ZROWS):
        pltpu.sync_copy(zero_v,
                        acc_sh.at[pl.ds(sid * ROWS_PER_TILE + z * ZROWS, ZROWS)])
    plsc.subcore_barrier()

    def load(g, p):
        base = wid * PER_W + g * CHUNK
        pltpu.sync_copy(tags_hbm.at[pl.ds(base, CHUNK)], idx[p])
        pltpu.sync_copy(seg_hbm.at[pl.ds(base // 128, SUBS)], seg[p])
        ref = idx[p]

        def tbody(k, carry):
            v = ref[pl.ds(k * 16, 16)]
            ref[pl.ds(k * 16, 16)] = ((v & -8192) | ((v & 1023) << 3)
                                      | ((v >> 10) & 7))
            return carry
        lax.fori_loop(0, CHUNK // 16, tbody, 0, unroll=8)

    def gather(p):
        return pltpu.make_async_copy(tab_hbm.at[idx[p]], rows[p], gsem)

    def scatter_start(p):
        for j in range(SUBS):
            pltpu.make_async_copy(rows[p].at[pl.ds(j * SUB, SUB)],
                                  acc_sh.at[seg[p].at[j]], ssem[p]).start(add=True)

    def scatter_drain(p):
        for j in range(SUBS):
            pltpu.make_async_copy(rows[p].at[pl.ds(j * SUB, SUB)],
                                  acc_sh.at[seg[p].at[j]], ssem[p]).wait()

    load(0, 0)
    gather(0).start()

    def body(g, carry):
        for p in (0, 1):
            @pl.when(g % 2 == p)
            def _():
                @pl.when(g > 0)
                def _():
                    scatter_drain(1 - p)

                @pl.when(g + 1 < CPW)
                def _():
                    load(g + 1, 1 - p)
                gather(p).wait()

                @pl.when(g + 1 < CPW)
                def _():
                    gather(1 - p).start()
                scatter_start(p)
        return carry
    lax.fori_loop(0, CPW, body, 0)
    scatter_drain((CPW - 1) % 2)

    plsc.subcore_barrier()
    pltpu.sync_copy(
        acc_sh.at[pl.ds(sid * ROWS_PER_TILE, ROWS_PER_TILE)],
        out_hbm.at[pl.ds(cid * B + sid * ROWS_PER_TILE, ROWS_PER_TILE)])


TBK = 8192
TGRID = (VOCAB + TBK - 1) // TBK
VOCABP = TGRID * TBK


def _tr_body(xt_ref, out_ref):
    y = jnp.concatenate(
        [xt_ref[:, 1024 * u:1024 * (u + 1)] for u in range(8)], axis=0)
    out_ref[...] = y.T


def _mlp_body(pa_ref, pb_ref, lab_ref, w1_ref, b1_ref, w2_ref, b2_ref,
              out_ref, loss_ref):
    i = pl.program_id(0)
    pooled = pa_ref[...] + pb_ref[...]
    h = jnp.maximum(
        jnp.dot(pooled, w1_ref[...], preferred_element_type=jnp.float32)
        + b1_ref[...], 0.0)
    logits = jnp.dot(h, w2_ref[...],
                     preferred_element_type=jnp.float32) + b2_ref[...]
    out_ref[...] = logits
    lab = lab_ref[...]
    cls = lax.broadcasted_iota(jnp.int32, (BLK, NUM_CLASS), 1)
    logit_lab = jnp.sum(jnp.where(cls == lab, logits, 0.0), axis=1,
                        keepdims=True)
    m = jnp.max(logits, axis=1, keepdims=True)
    lse = jnp.log(jnp.sum(jnp.exp(logits - m), axis=1, keepdims=True)) + m
    part = (jnp.sum(lse - logit_lab) / B).reshape(1, 1)

    @pl.when(i == 0)
    def _():
        loss_ref[...] = jnp.zeros_like(loss_ref)
    loss_ref[...] += part


def kernel(node_tags, segment_ids, labels, table, W1, b1, W2, b2):
    tab_lin = pl.pallas_call(
        _tr_body,
        grid=(TGRID,),
        in_specs=[pl.BlockSpec((EMBED, TBK), lambda i: (0, i))],
        out_specs=pl.BlockSpec((TBK // 8, 128), lambda i: (i, 0)),
        out_shape=jax.ShapeDtypeStruct((VOCABP // 8, 128), jnp.float32),
    )(table.T).reshape(VOCABP, EMBED)

    mesh = plsc.VectorSubcoreMesh(core_axis_name="c", subcore_axis_name="s",
                                  num_cores=NC, num_subcores=NS)
    sc_pool = pl.kernel(
        _sc_pool_body,
        out_type=jax.ShapeDtypeStruct((NC * B, EMBED), jnp.float32),
        mesh=mesh,
        scratch_types=[
            pltpu.VMEM((CHUNK,), jnp.int32),
            pltpu.VMEM((CHUNK,), jnp.int32),
            pltpu.VMEM((SUBS, SUB), jnp.int32),
            pltpu.VMEM((SUBS, SUB), jnp.int32),
            pltpu.VMEM((CHUNK, EMBED), jnp.float32),
            pltpu.VMEM((CHUNK, EMBED), jnp.float32),
            pltpu.VMEM((ZROWS, EMBED), jnp.float32),
            pltpu.VMEM_SHARED((B, EMBED), jnp.float32),
            pltpu.SemaphoreType.DMA,
            pltpu.SemaphoreType.DMA,
            pltpu.SemaphoreType.DMA,
        ],
        compiler_params=pltpu.CompilerParams(use_tc_tiling_on_sc=False),
    )
    part = sc_pool(node_tags, segment_ids.reshape(N128, 128), tab_lin)

    grid = (B // BLK,)
    logits, loss2 = pl.pallas_call(
        _mlp_body,
        grid=grid,
        in_specs=[
            pl.BlockSpec((BLK, EMBED), lambda i: (i, 0)),
            pl.BlockSpec((BLK, EMBED), lambda i: (i + B // BLK, 0)),
            pl.BlockSpec((BLK, 1), lambda i: (i, 0)),
            pl.BlockSpec((EMBED, HIDDEN), lambda i: (0, 0)),
            pl.BlockSpec((1, HIDDEN), lambda i: (0, 0)),
            pl.BlockSpec((HIDDEN, NUM_CLASS), lambda i: (0, 0)),
            pl.BlockSpec((1, NUM_CLASS), lambda i: (0, 0)),
        ],
        out_specs=[
            pl.BlockSpec((BLK, NUM_CLASS), lambda i: (i, 0)),
            pl.BlockSpec((1, 1), lambda i: (0, 0)),
        ],
        out_shape=[
            jax.ShapeDtypeStruct((B, NUM_CLASS), jnp.float32),
            jax.ShapeDtypeStruct((1, 1), jnp.float32),
        ],
    )(part, part, labels.reshape(B, 1), W1, b1.reshape(1, HIDDEN),
      W2, b2.reshape(1, NUM_CLASS))
    return logits, loss2[0, 0]


# speedup vs baseline: 2.8856x; 1.1421x over previous
import functools

import jax
import jax.numpy as jnp
from jax import lax
from jax.experimental import pallas as pl
from jax.experimental.pallas import tpu as pltpu, tpu_sc as plsc

VOCAB = 1000000
EMBED = 16
N = 1638400
B = 16384
HIDDEN = 128
NUM_CLASS = 10

NC = 2
NS = 16
NW = NC * NS
PER_W = N // NW
CHUNK = 2560
CPW = PER_W // CHUNK
SUB = 128
SUBS = CHUNK // SUB
ROWS_PER_TILE = B // NS
ZROWS = 128

BLK = 1024
N128 = N // 128


def _sc_pool_body(tags_hbm, seg_hbm, tab_hbm, out_hbm,
                  idx0, idx1, seg0, seg1, rows0, rows1, zero_v, acc_sh,
                  gsem, ssem0, ssem1, lsem):
    cid = lax.axis_index("c")
    sid = lax.axis_index("s")
    wid = cid * NS + sid
    idx = (idx0, idx1)
    seg = (seg0, seg1)
    rows = (rows0, rows1)
    ssem = (ssem0, ssem1)

    def zbody(i, carry):
        zero_v[i] = jnp.zeros((EMBED,), jnp.float32)
        return carry
    lax.fori_loop(0, ZROWS, zbody, 0)
    for z in range(ROWS_PER_TILE // ZROWS):
        pltpu.sync_copy(zero_v,
                        acc_sh.at[pl.ds(sid * ROWS_PER_TILE + z * ZROWS, ZROWS)])
    plsc.subcore_barrier()

    def load(g, p):
        base = wid * PER_W + g * CHUNK
        i_cp = pltpu.async_copy(tags_hbm.at[pl.ds(base, CHUNK)], idx[p], lsem)
        s_cp = pltpu.async_copy(seg_hbm.at[pl.ds(base // 128, SUBS)], seg[p],
                                lsem)
        i_cp.wait()
        s_cp.wait()
        ref = idx[p]

        def tbody(k, carry):
            v = ref[pl.ds(k * 16, 16)]
            ref[pl.ds(k * 16, 16)] = ((v & -TBK) | ((v & (TQ - 1)) << 3)
                                      | ((v >> 11) & 7))
            return carry
        lax.fori_loop(0, CHUNK // 16, tbody, 0, unroll=8)

    def gather(p):
        return pltpu.make_async_copy(tab_hbm.at[idx[p]], rows[p], gsem)

    def scatter_start(p):
        for j in range(SUBS):
            pltpu.make_async_copy(rows[p].at[pl.ds(j * SUB, SUB)],
                                  acc_sh.at[seg[p].at[j]], ssem[p]).start(add=True)

    def scatter_drain(p):
        for j in range(SUBS):
            pltpu.make_async_copy(rows[p].at[pl.ds(j * SUB, SUB)],
                                  acc_sh.at[seg[p].at[j]], ssem[p]).wait()

    load(0, 0)
    gather(0).start()

    def body(g, carry):
        for p in (0, 1):
            @pl.when(g % 2 == p)
            def _():
                @pl.when(g > 0)
                def _():
                    scatter_drain(1 - p)

                @pl.when(g + 1 < CPW)
                def _():
                    load(g + 1, 1 - p)
                gather(p).wait()

                @pl.when(g + 1 < CPW)
                def _():
                    gather(1 - p).start()
                scatter_start(p)
        return carry
    lax.fori_loop(0, CPW, body, 0)
    scatter_drain((CPW - 1) % 2)

    plsc.subcore_barrier()
    pltpu.sync_copy(
        acc_sh.at[pl.ds(sid * ROWS_PER_TILE, ROWS_PER_TILE)],
        out_hbm.at[pl.ds(cid * B + sid * ROWS_PER_TILE, ROWS_PER_TILE)])


TBK = 16384
TGRID = (VOCAB + TBK - 1) // TBK
VOCABP = TGRID * TBK
TQ = TBK // 8


def _tr_body(xt_ref, out_ref):
    y = jnp.concatenate(
        [xt_ref[:, TQ * u:TQ * (u + 1)] for u in range(8)], axis=0)
    out_ref[...] = y.T


def _mlp_body(pa_ref, pb_ref, lab_ref, w1_ref, b1_ref, w2_ref, b2_ref,
              out_ref, loss_ref):
    i = pl.program_id(0)
    pooled = pa_ref[...] + pb_ref[...]
    h = jnp.maximum(
        jnp.dot(pooled, w1_ref[...], preferred_element_type=jnp.float32)
        + b1_ref[...], 0.0)
    logits = jnp.dot(h, w2_ref[...],
                     preferred_element_type=jnp.float32) + b2_ref[...]
    out_ref[...] = logits
    lab = lab_ref[...]
    cls = lax.broadcasted_iota(jnp.int32, (BLK, NUM_CLASS), 1)
    logit_lab = jnp.sum(jnp.where(cls == lab, logits, 0.0), axis=1,
                        keepdims=True)
    m = jnp.max(logits, axis=1, keepdims=True)
    lse = jnp.log(jnp.sum(jnp.exp(logits - m), axis=1, keepdims=True)) + m
    part = (jnp.sum(lse - logit_lab) / B).reshape(1, 1)

    @pl.when(i == 0)
    def _():
        loss_ref[...] = jnp.zeros_like(loss_ref)
    loss_ref[...] += part


def kernel(node_tags, segment_ids, labels, table, W1, b1, W2, b2):
    tab_lin = pl.pallas_call(
        _tr_body,
        grid=(TGRID,),
        in_specs=[pl.BlockSpec((EMBED, TBK), lambda i: (0, i))],
        out_specs=pl.BlockSpec((TQ, 128), lambda i: (i, 0)),
        out_shape=jax.ShapeDtypeStruct((VOCABP // 8, 128), jnp.float32),
    )(table.T).reshape(VOCABP, EMBED)

    mesh = plsc.VectorSubcoreMesh(core_axis_name="c", subcore_axis_name="s",
                                  num_cores=NC, num_subcores=NS)
    sc_pool = pl.kernel(
        _sc_pool_body,
        out_type=jax.ShapeDtypeStruct((NC * B, EMBED), jnp.float32),
        mesh=mesh,
        scratch_types=[
            pltpu.VMEM((CHUNK,), jnp.int32),
            pltpu.VMEM((CHUNK,), jnp.int32),
            pltpu.VMEM((SUBS, SUB), jnp.int32),
            pltpu.VMEM((SUBS, SUB), jnp.int32),
            pltpu.VMEM((CHUNK, EMBED), jnp.float32),
            pltpu.VMEM((CHUNK, EMBED), jnp.float32),
            pltpu.VMEM((ZROWS, EMBED), jnp.float32),
            pltpu.VMEM_SHARED((B, EMBED), jnp.float32),
            pltpu.SemaphoreType.DMA,
            pltpu.SemaphoreType.DMA,
            pltpu.SemaphoreType.DMA,
            pltpu.SemaphoreType.DMA,
        ],
        compiler_params=pltpu.CompilerParams(use_tc_tiling_on_sc=False),
    )
    part = sc_pool(node_tags, segment_ids.reshape(N128, 128), tab_lin)

    grid = (B // BLK,)
    logits, loss2 = pl.pallas_call(
        _mlp_body,
        grid=grid,
        in_specs=[
            pl.BlockSpec((BLK, EMBED), lambda i: (i, 0)),
            pl.BlockSpec((BLK, EMBED), lambda i: (i + B // BLK, 0)),
            pl.BlockSpec((BLK, 1), lambda i: (i, 0)),
            pl.BlockSpec((EMBED, HIDDEN), lambda i: (0, 0)),
            pl.BlockSpec((1, HIDDEN), lambda i: (0, 0)),
            pl.BlockSpec((HIDDEN, NUM_CLASS), lambda i: (0, 0)),
            pl.BlockSpec((1, NUM_CLASS), lambda i: (0, 0)),
        ],
        out_specs=[
            pl.BlockSpec((BLK, NUM_CLASS), lambda i: (i, 0)),
            pl.BlockSpec((1, 1), lambda i: (0, 0)),
        ],
        out_shape=[
            jax.ShapeDtypeStruct((B, NUM_CLASS), jnp.float32),
            jax.ShapeDtypeStruct((1, 1), jnp.float32),
        ],
    )(part, part, labels.reshape(B, 1), W1, b1.reshape(1, HIDDEN),
      W2, b2.reshape(1, NUM_CLASS))
    return logits, loss2[0, 0]
